# baseline (speedup 1.0000x reference)
import functools

import jax
import jax.numpy as jnp
from jax.experimental import pallas as pl
from jax.experimental.pallas import tpu as pltpu


def _convp(z, w_ref, b_ref, HWg, G, Ci):
    bimg = z.shape[0]
    j8 = jax.lax.broadcasted_iota(jnp.int32, (bimg, HWg, Ci), 1) & 7
    zrow = jnp.zeros((bimg, 1, Ci), jnp.float32)
    dn = jnp.concatenate([zrow, z[:, :-1, (G - 1) * Ci:]], axis=1)
    up = jnp.concatenate([z[:, 1:, :Ci], zrow], axis=1)
    lpc = jnp.where(j8 != 0, dn, 0.0)
    rpc = jnp.where(j8 != 7, up, 0.0)
    s = jnp.concatenate([lpc, z, rpc], axis=-1)
    zpad = jnp.zeros((bimg, 8, (G + 2) * Ci), jnp.float32)
    sp = jnp.concatenate([zpad, s, zpad], axis=1)

    spf = sp.reshape(bimg * (HWg + 16), (G + 2) * Ci).astype(jnp.bfloat16)
    gco = w_ref.shape[-1]
    acc = None
    for kh in range(3):
        part = jnp.dot(spf, w_ref[kh],
                       preferred_element_type=jnp.float32)
        part = part.reshape(bimg, HWg + 16, gco)[:, kh * 8:kh * 8 + HWg]
        acc = part if acc is None else acc + part
    return jnp.maximum(acc + b_ref[...], 0.0)


def _mxu_stats(r, sum_ref, sq_ref):
    sum_ref[...] = jnp.sum(r, axis=0, keepdims=True)
    sq_ref[...] = jnp.sum(r * r, axis=0, keepdims=True)
    return r.astype(jnp.bfloat16)


def _pool_affine(x_ref, s_ref, t_ref, C):
    y = x_ref[...].astype(jnp.float32) * s_ref[...] + t_ref[...]
    v = jnp.maximum(y[:, 0], y[:, 1])
    return jnp.maximum(v[:, :, :C], v[:, :, C:])


def _stage_a_kernel(x_ref, w1_ref, b1_ref, w2_ref, b2_ref,
                    o_ref, sum_ref, sq_ref, *, bimg):
    m = bimg * 256
    acc = jnp.dot(x_ref[...].reshape(m, 128), w1_ref[...],
                  preferred_element_type=jnp.float32)
    z1 = jnp.maximum(acc + b1_ref[...], 0.0).reshape(bimg, 256, 128)

    r = _convp(z1, w2_ref, b2_ref, HWg=256, G=4, Ci=32)
    rb = _mxu_stats(r.reshape(m, 256), sum_ref, sq_ref)
    o_ref[...] = rb.reshape(bimg, 256, 256)


def _pool_kernel(x_ref, s_ref, t_ref, o_ref):
    o_ref[...] = _pool_affine(x_ref, s_ref, t_ref, 64).astype(o_ref.dtype)


def _stage_b2_kernel(x_ref, w3_ref, b3_ref, w4_ref, b4_ref,
                     o_ref, sum_ref, sq_ref, *, bimg):
    p = x_ref[...].astype(jnp.float32)
    z3 = _convp(p, w3_ref, b3_ref, HWg=128, G=2, Ci=64)
    r = _convp(z3, w4_ref, b4_ref, HWg=128, G=2, Ci=128)
    rb = _mxu_stats(r.reshape(bimg * 128, 256), sum_ref, sq_ref)
    o_ref[...] = rb.reshape(bimg, 128, 256)


def _stage_c_kernel(x_ref, s_ref, t_ref, w5_ref, b5_ref, w6_ref, b6_ref,
                    o_ref, sum_ref, sq_ref, *, bimg):
    vv = _pool_affine(x_ref, s_ref, t_ref, 128)
    x5 = vv.reshape(bimg, 64, 128)
    y5 = _convp(x5, w5_ref, b5_ref, HWg=64, G=1, Ci=128)
    r = _convp(y5, w6_ref, b6_ref, HWg=64, G=1, Ci=256)
    rb = _mxu_stats(r.reshape(bimg * 64, 256), sum_ref, sq_ref)
    o_ref[...] = rb.reshape(bimg, 64, 256)


def _stage_d_kernel(x_ref, s_ref, t_ref, fw1_ref, fb1_ref, fw2_ref, fb2_ref,
                    fw3_ref, fb3_ref, o_ref, *, bimg):
    vv = _pool_affine(x_ref, s_ref, t_ref, 256)
    x4 = vv.reshape(bimg, 4, 4, 256).astype(jnp.bfloat16)

    acc = fb1_ref[...]
    for a in range(4):
        for b in range(4):
            wslab = fw1_ref[pl.ds((a * 4 + b) * 256, 256), :]
            acc = acc + jnp.dot(x4[:, a, b, :], wslab,
                                preferred_element_type=jnp.float32)
    h1 = jnp.maximum(acc, 0.0).astype(jnp.bfloat16)
    h2 = jnp.maximum(
        jnp.dot(h1, fw2_ref[...], preferred_element_type=jnp.float32)
        + fb2_ref[...], 0.0).astype(jnp.bfloat16)
    o_ref[...] = (jnp.dot(h2, fw3_ref[...],
                          preferred_element_type=jnp.float32) + fb3_ref[...])


def _div_leq(n, cap):
    cap = max(1, min(n, cap))
    for d in range(cap, 0, -1):
        if n % d == 0:
            return d
    return 1


def _bn_affine(sums, sqs, gamma, beta, count, groups):
    c = gamma.shape[0]
    s = jnp.sum(sums.reshape(-1, groups, c), axis=(0, 1))
    q = jnp.sum(sqs.reshape(-1, groups, c), axis=(0, 1))
    mean = s / count
    var = q / count - mean * mean
    inv = jax.lax.rsqrt(var + 1e-5)
    scale = gamma * inv
    shift = beta - mean * scale
    s2 = jnp.concatenate([scale, scale]).reshape(1, 1, 1, 2 * c)
    t2 = jnp.concatenate([shift, shift]).reshape(1, 1, 1, 2 * c)
    return s2.astype(jnp.float32), t2.astype(jnp.float32)


def _wpack(w_hwio, G):
    kh, kw, ci, co = w_hwio.shape
    out = jnp.zeros((kh, G + 2, ci, G, co), w_hwio.dtype)
    for t in range(G):
        for k in range(kw):
            out = out.at[:, t + k, :, t, :].add(w_hwio[:, k])
    return out.reshape(kh, (G + 2) * ci, G * co).astype(jnp.bfloat16)


def _w9(w_hwio):
    kh, kw, cin, cout = w_hwio.shape
    return w_hwio.reshape(kh, kw * cin, cout).astype(jnp.bfloat16)


@jax.jit
def _forward(x_nchw, w1, b1, w2, b2, g1, bt1, w3, b3, w4, b4, g2, bt2,
             w5, b5, w6, b6, g3, bt3, fw1, fb1, fw2, fb2, fw3, fb3):
    n = x_nchw.shape[0]
    f32 = jnp.float32

    xt = jnp.transpose(x_nchw, (0, 2, 3, 1)).astype(jnp.bfloat16)
    xp = jnp.pad(xt, ((0, 0), (1, 1), (1, 1), (0, 0)))
    cols = jnp.concatenate(
        [xp[:, kh:kh + 32, kw:kw + 32, :] for kh in range(3)
         for kw in range(3)], axis=-1)
    cols = jnp.pad(cols, ((0, 0), (0, 0), (0, 0), (0, 5)))
    cols = cols.reshape(n, 256, 128)

    w1pad = jnp.pad(w1.transpose(0, 1, 2, 3).reshape(9, 3, 32).reshape(27, 32),
                    ((0, 5), (0, 0)))
    eye4 = jnp.eye(4, dtype=f32)
    w1q = (eye4[:, None, :, None] * w1pad[None, :, None, :])
    w1q = w1q.reshape(128, 128).astype(jnp.bfloat16)
    b1q = jnp.tile(b1, 4).reshape(1, 128).astype(f32)

    w2q = _wpack(w2, 4)
    b2q = jnp.tile(b2, 4).reshape(1, 256).astype(f32)
    w3p = _wpack(w3, 2)
    b3p = jnp.tile(b3, 2).reshape(1, 256).astype(f32)
    w4p = _wpack(w4, 2)
    b4p = jnp.tile(b4, 2).reshape(1, 256).astype(f32)
    w5r, w6r = _w9(w5), _w9(w6)
    b5r = b5.reshape(1, 256).astype(f32)
    b6r = b6.reshape(1, 256).astype(f32)

    fw1r = fw1.reshape(256, 16, 1024).transpose(1, 0, 2).reshape(4096, 1024)
    fw1r = fw1r.astype(jnp.bfloat16)
    fw2r = fw2.astype(jnp.bfloat16)
    fw3r = jnp.pad(fw3, ((0, 0), (0, 118))).astype(jnp.bfloat16)
    fb1r = fb1.reshape(1, 1024).astype(f32)
    fb2r = fb2.reshape(1, 512).astype(f32)
    fb3r = jnp.pad(fb3, ((0, 118),)).reshape(1, 128).astype(f32)

    ba = _div_leq(n, 64)
    ga = n // ba
    y2, s1, q1 = pl.pallas_call(
        functools.partial(_stage_a_kernel, bimg=ba),
        out_shape=[
            jax.ShapeDtypeStruct((n, 256, 256), jnp.bfloat16),
            jax.ShapeDtypeStruct((ga, 1, 256), f32),
            jax.ShapeDtypeStruct((ga, 1, 256), f32),
        ],
        grid_spec=pltpu.PrefetchScalarGridSpec(
            num_scalar_prefetch=0,
            grid=(ga,),
            in_specs=[
                pl.BlockSpec((ba, 256, 128), lambda i: (i, 0, 0)),
                pl.BlockSpec((128, 128), lambda i: (0, 0)),
                pl.BlockSpec((1, 128), lambda i: (0, 0)),
                pl.BlockSpec((3, 192, 256), lambda i: (0, 0, 0)),
                pl.BlockSpec((1, 256), lambda i: (0, 0)),
            ],
            out_specs=[
                pl.BlockSpec((ba, 256, 256), lambda i: (i, 0, 0)),
                pl.BlockSpec((None, 1, 256), lambda i: (i, 0, 0)),
                pl.BlockSpec((None, 1, 256), lambda i: (i, 0, 0)),
            ],
        ),
        compiler_params=pltpu.CompilerParams(
            dimension_semantics=("parallel",)),
    )(cols, w1q, b1q, w2q, b2q)

    s2a, t2a = _bn_affine(s1, q1, g1, bt1, float(n * 1024), 4)

    rows1 = n * 16
    bm = _div_leq(rows1, 1024)
    xb = y2.reshape(rows1, 2, 16, 128)
    p1 = pl.pallas_call(
        _pool_kernel,
        out_shape=jax.ShapeDtypeStruct((rows1, 16, 64), jnp.bfloat16),
        grid_spec=pltpu.PrefetchScalarGridSpec(
            num_scalar_prefetch=0,
            grid=(rows1 // bm,),
            in_specs=[
                pl.BlockSpec((bm, 2, 16, 128), lambda i: (i, 0, 0, 0)),
                pl.BlockSpec((1, 1, 1, 128), lambda i: (0, 0, 0, 0)),
                pl.BlockSpec((1, 1, 1, 128), lambda i: (0, 0, 0, 0)),
            ],
            out_specs=pl.BlockSpec((bm, 16, 64), lambda i: (i, 0, 0)),
        ),
        compiler_params=pltpu.CompilerParams(
            dimension_semantics=("parallel",)),
    )(xb, s2a, t2a)

    bb = _div_leq(n, 64)
    gb = n // bb
    xb2 = p1.reshape(n, 128, 128)
    y4, s2_, q2_ = pl.pallas_call(
        functools.partial(_stage_b2_kernel, bimg=bb),
        out_shape=[
            jax.ShapeDtypeStruct((n, 128, 256), jnp.bfloat16),
            jax.ShapeDtypeStruct((gb, 1, 256), f32),
            jax.ShapeDtypeStruct((gb, 1, 256), f32),
        ],
        grid_spec=pltpu.PrefetchScalarGridSpec(
            num_scalar_prefetch=0,
            grid=(gb,),
            in_specs=[
                pl.BlockSpec((bb, 128, 128), lambda i: (i, 0, 0)),
                pl.BlockSpec((3, 256, 256), lambda i: (0, 0, 0)),
                pl.BlockSpec((1, 256), lambda i: (0, 0)),
                pl.BlockSpec((3, 512, 256), lambda i: (0, 0, 0)),
                pl.BlockSpec((1, 256), lambda i: (0, 0)),
            ],
            out_specs=[
                pl.BlockSpec((bb, 128, 256), lambda i: (i, 0, 0)),
                pl.BlockSpec((None, 1, 256), lambda i: (i, 0, 0)),
                pl.BlockSpec((None, 1, 256), lambda i: (i, 0, 0)),
            ],
        ),
        compiler_params=pltpu.CompilerParams(
            dimension_semantics=("parallel",)),
    )(xb2, w3p, b3p, w4p, b4p)

    s2b, t2b = _bn_affine(s2_, q2_, g2, bt2, float(n * 256), 2)

    bc = _div_leq(n, 64)
    gc = n // bc
    xc = y4.reshape(n * 8, 2, 8, 256)
    y6, s3_, q3_ = pl.pallas_call(
        functools.partial(_stage_c_kernel, bimg=bc),
        out_shape=[
            jax.ShapeDtypeStruct((n, 64, 256), jnp.bfloat16),
            jax.ShapeDtypeStruct((gc, 1, 256), f32),
            jax.ShapeDtypeStruct((gc, 1, 256), f32),
        ],
        grid_spec=pltpu.PrefetchScalarGridSpec(
            num_scalar_prefetch=0,
            grid=(gc,),
            in_specs=[
                pl.BlockSpec((bc * 8, 2, 8, 256), lambda i: (i, 0, 0, 0)),
                pl.BlockSpec((1, 1, 1, 256), lambda i: (0, 0, 0, 0)),
                pl.BlockSpec((1, 1, 1, 256), lambda i: (0, 0, 0, 0)),
                pl.BlockSpec((3, 384, 256), lambda i: (0, 0, 0)),
                pl.BlockSpec((1, 256), lambda i: (0, 0)),
                pl.BlockSpec((3, 768, 256), lambda i: (0, 0, 0)),
                pl.BlockSpec((1, 256), lambda i: (0, 0)),
            ],
            out_specs=[
                pl.BlockSpec((bc, 64, 256), lambda i: (i, 0, 0)),
                pl.BlockSpec((None, 1, 256), lambda i: (i, 0, 0)),
                pl.BlockSpec((None, 1, 256), lambda i: (i, 0, 0)),
            ],
        ),
        compiler_params=pltpu.CompilerParams(
            dimension_semantics=("parallel",)),
    )(xc, s2b, t2b, w5r, b5r, w6r, b6r)

    s2c, t2c = _bn_affine(s3_, q3_, g3, bt3, float(n * 64), 1)

    bd = _div_leq(n, 512)
    gd = n // bd
    xd = y6.reshape(n * 4, 2, 4, 512)
    out = pl.pallas_call(
        functools.partial(_stage_d_kernel, bimg=bd),
        out_shape=jax.ShapeDtypeStruct((n, 128), f32),
        grid_spec=pltpu.PrefetchScalarGridSpec(
            num_scalar_prefetch=0,
            grid=(gd,),
            in_specs=[
                pl.BlockSpec((bd * 4, 2, 4, 512), lambda i: (i, 0, 0, 0)),
                pl.BlockSpec((1, 1, 1, 512), lambda i: (0, 0, 0, 0)),
                pl.BlockSpec((1, 1, 1, 512), lambda i: (0, 0, 0, 0)),
                pl.BlockSpec((4096, 1024), lambda i: (0, 0)),
                pl.BlockSpec((1, 1024), lambda i: (0, 0)),
                pl.BlockSpec((1024, 512), lambda i: (0, 0)),
                pl.BlockSpec((1, 512), lambda i: (0, 0)),
                pl.BlockSpec((512, 128), lambda i: (0, 0)),
                pl.BlockSpec((1, 128), lambda i: (0, 0)),
            ],
            out_specs=pl.BlockSpec((bd, 128), lambda i: (i, 0)),
        ),
        compiler_params=pltpu.CompilerParams(
            dimension_semantics=("parallel",)),
    )(xd, s2c, t2c, fw1r, fb1r, fw2r, fb2r, fw3r, fb3r)

    return out[:, :10]


def kernel(x_nchw, w1, b1, w2, b2, g1, bt1, w3, b3, w4, b4, g2, bt2,
           w5, b5, w6, b6, g3, bt3, fw1, fb1, fw2, fb2, fw3, fb3):
    return _forward(x_nchw, w1, b1, w2, b2, g1, bt1, w3, b3, w4, b4, g2, bt2,
                    w5, b5, w6, b6, g3, bt3, fw1, fb1, fw2, fb2, fw3, fb3)


# speedup vs baseline: 1.0063x; 1.0063x over previous
import functools

import jax
import jax.numpy as jnp
from jax.experimental import pallas as pl
from jax.experimental.pallas import tpu as pltpu


def _convp(z, w_ref, b_ref, HWg, G, Ci):
    bimg = z.shape[0]
    j8 = jax.lax.broadcasted_iota(jnp.int32, (bimg, HWg, Ci), 1) & 7
    zrow = jnp.zeros((bimg, 1, Ci), jnp.float32)
    dn = jnp.concatenate([zrow, z[:, :-1, (G - 1) * Ci:]], axis=1)
    up = jnp.concatenate([z[:, 1:, :Ci], zrow], axis=1)
    lpc = jnp.where(j8 != 0, dn, 0.0)
    rpc = jnp.where(j8 != 7, up, 0.0)
    s = jnp.concatenate([lpc, z, rpc], axis=-1)
    zpad = jnp.zeros((bimg, 8, (G + 2) * Ci), jnp.float32)
    sp = jnp.concatenate([zpad, s, zpad], axis=1)

    spf = sp.reshape(bimg * (HWg + 16), (G + 2) * Ci).astype(jnp.bfloat16)
    gco = w_ref.shape[-1]
    acc = None
    for kh in range(3):
        part = jnp.dot(spf, w_ref[kh],
                       preferred_element_type=jnp.float32)
        part = part.reshape(bimg, HWg + 16, gco)[:, kh * 8:kh * 8 + HWg]
        acc = part if acc is None else acc + part
    return jnp.maximum(acc + b_ref[...], 0.0)


def _mxu_stats(r, sum_ref, sq_ref):
    sum_ref[...] = jnp.sum(r, axis=0, keepdims=True)
    sq_ref[...] = jnp.sum(r * r, axis=0, keepdims=True)
    return r.astype(jnp.bfloat16)


def _pool_affine(x_ref, s_ref, t_ref, C):
    y = x_ref[...].astype(jnp.float32) * s_ref[...] + t_ref[...]
    v = jnp.maximum(y[:, 0], y[:, 1])
    return jnp.maximum(v[:, :, :C], v[:, :, C:])


def _stage_a_kernel(x_ref, w1_ref, b1_ref, w2_ref, b2_ref,
                    o_ref, sum_ref, sq_ref, *, bimg):
    m = bimg * 256
    acc = jnp.dot(x_ref[...].reshape(m, 128), w1_ref[...],
                  preferred_element_type=jnp.float32)
    z1 = jnp.maximum(acc + b1_ref[...], 0.0).reshape(bimg, 256, 128)

    r = _convp(z1, w2_ref, b2_ref, HWg=256, G=4, Ci=32)
    rb = _mxu_stats(r.reshape(m, 256), sum_ref, sq_ref)
    o_ref[...] = rb.reshape(bimg, 256, 256)


def _pool_kernel(x_ref, s_ref, t_ref, o_ref):
    o_ref[...] = _pool_affine(x_ref, s_ref, t_ref, 64).astype(o_ref.dtype)


def _stage_b2_kernel(x_ref, w3_ref, b3_ref, w4_ref, b4_ref,
                     o_ref, sum_ref, sq_ref, *, bimg):
    p = x_ref[...].astype(jnp.float32)
    z3 = _convp(p, w3_ref, b3_ref, HWg=128, G=2, Ci=64)
    r = _convp(z3, w4_ref, b4_ref, HWg=128, G=2, Ci=128)
    rb = _mxu_stats(r.reshape(bimg * 128, 256), sum_ref, sq_ref)
    o_ref[...] = rb.reshape(bimg, 128, 256)


def _stage_c_kernel(x_ref, s_ref, t_ref, w5_ref, b5_ref, w6_ref, b6_ref,
                    o_ref, sum_ref, sq_ref, *, bimg):
    vv = _pool_affine(x_ref, s_ref, t_ref, 128)
    x5 = vv.reshape(bimg, 64, 128)
    y5 = _convp(x5, w5_ref, b5_ref, HWg=64, G=1, Ci=128)
    r = _convp(y5, w6_ref, b6_ref, HWg=64, G=1, Ci=256)
    rb = _mxu_stats(r.reshape(bimg * 64, 256), sum_ref, sq_ref)
    o_ref[...] = rb.reshape(bimg, 64, 256)


def _stage_d_kernel(x_ref, s_ref, t_ref, fw1_ref, fb1_ref, fw2_ref, fb2_ref,
                    fw3_ref, fb3_ref, o_ref, *, bimg):
    vv = _pool_affine(x_ref, s_ref, t_ref, 256)
    x4 = vv.reshape(bimg, 4, 4, 256).astype(jnp.bfloat16)

    acc = fb1_ref[...]
    for a in range(4):
        for b in range(4):
            wslab = fw1_ref[pl.ds((a * 4 + b) * 256, 256), :]
            acc = acc + jnp.dot(x4[:, a, b, :], wslab,
                                preferred_element_type=jnp.float32)
    h1 = jnp.maximum(acc, 0.0).astype(jnp.bfloat16)
    h2 = jnp.maximum(
        jnp.dot(h1, fw2_ref[...], preferred_element_type=jnp.float32)
        + fb2_ref[...], 0.0).astype(jnp.bfloat16)
    o_ref[...] = (jnp.dot(h2, fw3_ref[...],
                          preferred_element_type=jnp.float32) + fb3_ref[...])


def _div_leq(n, cap):
    cap = max(1, min(n, cap))
    for d in range(cap, 0, -1):
        if n % d == 0:
            return d
    return 1


def _bn_affine(sums, sqs, gamma, beta, count, groups):
    c = gamma.shape[0]
    s = jnp.sum(sums.reshape(-1, groups, c), axis=(0, 1))
    q = jnp.sum(sqs.reshape(-1, groups, c), axis=(0, 1))
    mean = s / count
    var = q / count - mean * mean
    inv = jax.lax.rsqrt(var + 1e-5)
    scale = gamma * inv
    shift = beta - mean * scale
    s2 = jnp.concatenate([scale, scale]).reshape(1, 1, 1, 2 * c)
    t2 = jnp.concatenate([shift, shift]).reshape(1, 1, 1, 2 * c)
    return s2.astype(jnp.float32), t2.astype(jnp.float32)


def _wpack(w_hwio, G):
    kh, kw, ci, co = w_hwio.shape
    out = jnp.zeros((kh, G + 2, ci, G, co), w_hwio.dtype)
    for t in range(G):
        for k in range(kw):
            out = out.at[:, t + k, :, t, :].add(w_hwio[:, k])
    return out.reshape(kh, (G + 2) * ci, G * co).astype(jnp.bfloat16)


def _w9(w_hwio):
    kh, kw, cin, cout = w_hwio.shape
    return w_hwio.reshape(kh, kw * cin, cout).astype(jnp.bfloat16)


@jax.jit
def _forward(x_nchw, w1, b1, w2, b2, g1, bt1, w3, b3, w4, b4, g2, bt2,
             w5, b5, w6, b6, g3, bt3, fw1, fb1, fw2, fb2, fw3, fb3):
    n = x_nchw.shape[0]
    f32 = jnp.float32

    xt = jnp.transpose(x_nchw, (0, 2, 3, 1)).astype(jnp.bfloat16)
    xp = jnp.pad(xt, ((0, 0), (1, 1), (1, 1), (0, 0)))
    cols = jnp.concatenate(
        [xp[:, kh:kh + 32, kw:kw + 32, :] for kh in range(3)
         for kw in range(3)], axis=-1)
    cols = jnp.pad(cols, ((0, 0), (0, 0), (0, 0), (0, 5)))
    cols = cols.reshape(n, 256, 128)

    w1pad = jnp.pad(w1.transpose(0, 1, 2, 3).reshape(9, 3, 32).reshape(27, 32),
                    ((0, 5), (0, 0)))
    eye4 = jnp.eye(4, dtype=f32)
    w1q = (eye4[:, None, :, None] * w1pad[None, :, None, :])
    w1q = w1q.reshape(128, 128).astype(jnp.bfloat16)
    b1q = jnp.tile(b1, 4).reshape(1, 128).astype(f32)

    w2q = _wpack(w2, 4)
    b2q = jnp.tile(b2, 4).reshape(1, 256).astype(f32)
    w3p = _wpack(w3, 2)
    b3p = jnp.tile(b3, 2).reshape(1, 256).astype(f32)
    w4p = _wpack(w4, 2)
    b4p = jnp.tile(b4, 2).reshape(1, 256).astype(f32)
    w5r, w6r = _w9(w5), _w9(w6)
    b5r = b5.reshape(1, 256).astype(f32)
    b6r = b6.reshape(1, 256).astype(f32)

    fw1r = fw1.reshape(256, 16, 1024).transpose(1, 0, 2).reshape(4096, 1024)
    fw1r = fw1r.astype(jnp.bfloat16)
    fw2r = fw2.astype(jnp.bfloat16)
    fw3r = jnp.pad(fw3, ((0, 0), (0, 118))).astype(jnp.bfloat16)
    fb1r = fb1.reshape(1, 1024).astype(f32)
    fb2r = fb2.reshape(1, 512).astype(f32)
    fb3r = jnp.pad(fb3, ((0, 118),)).reshape(1, 128).astype(f32)

    ba = _div_leq(n, 64)
    ga = n // ba
    y2, s1, q1 = pl.pallas_call(
        functools.partial(_stage_a_kernel, bimg=ba),
        out_shape=[
            jax.ShapeDtypeStruct((n, 256, 256), jnp.bfloat16),
            jax.ShapeDtypeStruct((ga, 1, 256), f32),
            jax.ShapeDtypeStruct((ga, 1, 256), f32),
        ],
        grid_spec=pltpu.PrefetchScalarGridSpec(
            num_scalar_prefetch=0,
            grid=(ga,),
            in_specs=[
                pl.BlockSpec((ba, 256, 128), lambda i: (i, 0, 0)),
                pl.BlockSpec((128, 128), lambda i: (0, 0)),
                pl.BlockSpec((1, 128), lambda i: (0, 0)),
                pl.BlockSpec((3, 192, 256), lambda i: (0, 0, 0)),
                pl.BlockSpec((1, 256), lambda i: (0, 0)),
            ],
            out_specs=[
                pl.BlockSpec((ba, 256, 256), lambda i: (i, 0, 0)),
                pl.BlockSpec((None, 1, 256), lambda i: (i, 0, 0)),
                pl.BlockSpec((None, 1, 256), lambda i: (i, 0, 0)),
            ],
        ),
        compiler_params=pltpu.CompilerParams(
            dimension_semantics=("parallel",)),
    )(cols, w1q, b1q, w2q, b2q)

    s2a, t2a = _bn_affine(s1, q1, g1, bt1, float(n * 1024), 4)

    rows1 = n * 16
    bm = _div_leq(rows1, 1024)
    xb = y2.reshape(rows1, 2, 16, 128)
    p1 = pl.pallas_call(
        _pool_kernel,
        out_shape=jax.ShapeDtypeStruct((rows1, 16, 64), jnp.bfloat16),
        grid_spec=pltpu.PrefetchScalarGridSpec(
            num_scalar_prefetch=0,
            grid=(rows1 // bm,),
            in_specs=[
                pl.BlockSpec((bm, 2, 16, 128), lambda i: (i, 0, 0, 0)),
                pl.BlockSpec((1, 1, 1, 128), lambda i: (0, 0, 0, 0)),
                pl.BlockSpec((1, 1, 1, 128), lambda i: (0, 0, 0, 0)),
            ],
            out_specs=pl.BlockSpec((bm, 16, 64), lambda i: (i, 0, 0)),
        ),
        compiler_params=pltpu.CompilerParams(
            dimension_semantics=("parallel",)),
    )(xb, s2a, t2a)

    bb = _div_leq(n, 64)
    gb = n // bb
    xb2 = p1.reshape(n, 128, 128)
    y4, s2_, q2_ = pl.pallas_call(
        functools.partial(_stage_b2_kernel, bimg=bb),
        out_shape=[
            jax.ShapeDtypeStruct((n, 128, 256), jnp.bfloat16),
            jax.ShapeDtypeStruct((gb, 1, 256), f32),
            jax.ShapeDtypeStruct((gb, 1, 256), f32),
        ],
        grid_spec=pltpu.PrefetchScalarGridSpec(
            num_scalar_prefetch=0,
            grid=(gb,),
            in_specs=[
                pl.BlockSpec((bb, 128, 128), lambda i: (i, 0, 0)),
                pl.BlockSpec((3, 256, 256), lambda i: (0, 0, 0)),
                pl.BlockSpec((1, 256), lambda i: (0, 0)),
                pl.BlockSpec((3, 512, 256), lambda i: (0, 0, 0)),
                pl.BlockSpec((1, 256), lambda i: (0, 0)),
            ],
            out_specs=[
                pl.BlockSpec((bb, 128, 256), lambda i: (i, 0, 0)),
                pl.BlockSpec((None, 1, 256), lambda i: (i, 0, 0)),
                pl.BlockSpec((None, 1, 256), lambda i: (i, 0, 0)),
            ],
        ),
        compiler_params=pltpu.CompilerParams(
            dimension_semantics=("parallel",)),
    )(xb2, w3p, b3p, w4p, b4p)

    s2b, t2b = _bn_affine(s2_, q2_, g2, bt2, float(n * 256), 2)

    bc = _div_leq(n, 64)
    gc = n // bc
    xc = y4.reshape(n * 8, 2, 8, 256)
    y6, s3_, q3_ = pl.pallas_call(
        functools.partial(_stage_c_kernel, bimg=bc),
        out_shape=[
            jax.ShapeDtypeStruct((n, 64, 256), jnp.bfloat16),
            jax.ShapeDtypeStruct((gc, 1, 256), f32),
            jax.ShapeDtypeStruct((gc, 1, 256), f32),
        ],
        grid_spec=pltpu.PrefetchScalarGridSpec(
            num_scalar_prefetch=0,
            grid=(gc,),
            in_specs=[
                pl.BlockSpec((bc * 8, 2, 8, 256), lambda i: (i, 0, 0, 0)),
                pl.BlockSpec((1, 1, 1, 256), lambda i: (0, 0, 0, 0)),
                pl.BlockSpec((1, 1, 1, 256), lambda i: (0, 0, 0, 0)),
                pl.BlockSpec((3, 384, 256), lambda i: (0, 0, 0)),
                pl.BlockSpec((1, 256), lambda i: (0, 0)),
                pl.BlockSpec((3, 768, 256), lambda i: (0, 0, 0)),
                pl.BlockSpec((1, 256), lambda i: (0, 0)),
            ],
            out_specs=[
                pl.BlockSpec((bc, 64, 256), lambda i: (i, 0, 0)),
                pl.BlockSpec((None, 1, 256), lambda i: (i, 0, 0)),
                pl.BlockSpec((None, 1, 256), lambda i: (i, 0, 0)),
            ],
        ),
        compiler_params=pltpu.CompilerParams(
            dimension_semantics=("parallel",)),
    )(xc, s2b, t2b, w5r, b5r, w6r, b6r)

    s2c, t2c = _bn_affine(s3_, q3_, g3, bt3, float(n * 64), 1)

    bd = _div_leq(n, 256)
    gd = n // bd
    xd = y6.reshape(n * 4, 2, 4, 512)
    out = pl.pallas_call(
        functools.partial(_stage_d_kernel, bimg=bd),
        out_shape=jax.ShapeDtypeStruct((n, 128), f32),
        grid_spec=pltpu.PrefetchScalarGridSpec(
            num_scalar_prefetch=0,
            grid=(gd,),
            in_specs=[
                pl.BlockSpec((bd * 4, 2, 4, 512), lambda i: (i, 0, 0, 0)),
                pl.BlockSpec((1, 1, 1, 512), lambda i: (0, 0, 0, 0)),
                pl.BlockSpec((1, 1, 1, 512), lambda i: (0, 0, 0, 0)),
                pl.BlockSpec((4096, 1024), lambda i: (0, 0)),
                pl.BlockSpec((1, 1024), lambda i: (0, 0)),
                pl.BlockSpec((1024, 512), lambda i: (0, 0)),
                pl.BlockSpec((1, 512), lambda i: (0, 0)),
                pl.BlockSpec((512, 128), lambda i: (0, 0)),
                pl.BlockSpec((1, 128), lambda i: (0, 0)),
            ],
            out_specs=pl.BlockSpec((bd, 128), lambda i: (i, 0)),
        ),
        compiler_params=pltpu.CompilerParams(
            dimension_semantics=("parallel",)),
    )(xd, s2c, t2c, fw1r, fb1r, fw2r, fb2r, fw3r, fb3r)

    return out[:, :10]


def kernel(x_nchw, w1, b1, w2, b2, g1, bt1, w3, b3, w4, b4, g2, bt2,
           w5, b5, w6, b6, g3, bt3, fw1, fb1, fw2, fb2, fw3, fb3):
    return _forward(x_nchw, w1, b1, w2, b2, g1, bt1, w3, b3, w4, b4, g2, bt2,
                    w5, b5, w6, b6, g3, bt3, fw1, fb1, fw2, fb2, fw3, fb3)
